# baseline (speedup 1.0000x reference)
import functools

import jax
import jax.numpy as jnp
from jax import lax
from jax.experimental import pallas as pl
from jax.experimental.pallas import tpu as pltpu
from jax.experimental.pallas import tpu_sc as plsc

BATCH = 2
SEQ = 2048
D_MODEL = 1024
N_HEAD = 16
D_K = 64
WIN = 8
M_SLOTS = 2 * WIN + 1
EPS = 1e-6

PADL = 64
SP = SEQ + 2 * PADL
T2 = SEQ // 4
CH = 256
KW = CH + 2 * WIN
NCH = SEQ // CH


def _proj_kernel(x_ref, wqt_ref, wkt_ref, wvt_ref, qt_ref, kt_ref, vt_ref):
    x = x_ref[...]
    nt = (((1,), (1,)), ((), ()))
    q = lax.dot_general(wqt_ref[...], x, nt, preferred_element_type=jnp.float32)
    qt_ref[...] = q * jnp.float32(1.0 / 8.0)
    kt_ref[...] = lax.dot_general(wkt_ref[...], x, nt,
                                  preferred_element_type=jnp.float32)
    vt_ref[...] = lax.dot_general(wvt_ref[...], x, nt,
                                  preferred_element_type=jnp.float32)


def _sc_attention(qt_hbm, kt_hbm, vt_hbm, att_hbm, q_v, k_v, v_v, o_v, *,
                  half):
    wid = lax.axis_index("s") * 2 + lax.axis_index("c")
    sub = wid % 2
    h = wid // 2
    row0 = h * D_K

    def chunk_body(c, _):
        local = sub * (NCH // 4) + c
        chunk = half * (NCH // 2) + local
        c0 = PADL + chunk * CH
        pltpu.sync_copy(qt_hbm.at[pl.ds(row0, D_K), pl.ds(c0, CH)], q_v)
        pltpu.sync_copy(kt_hbm.at[pl.ds(row0, D_K), pl.ds(c0 - WIN, KW)], k_v)
        pltpu.sync_copy(vt_hbm.at[pl.ds(row0, D_K), pl.ds(c0 - WIN, KW)], v_v)

        def group_body(g, _):
            base = g * 16
            kbase = g * 16

            def score_d(i, accs):
                for u in range(4):
                    d = i * 4 + u
                    qv = q_v[d, pl.ds(base, 16)]
                    accs = tuple(
                        accs[m] + qv * k_v[d, pl.ds(kbase + m, 16)]
                        for m in range(M_SLOTS))
                return accs

            zeros = tuple(jnp.zeros((16,), jnp.float32) for _ in range(M_SLOTS))
            s = lax.fori_loop(0, D_K // 4, score_d, zeros)

            def _tree(vals, op):
                vals = list(vals)
                while len(vals) > 1:
                    nxt = [op(vals[i], vals[i + 1])
                           for i in range(0, len(vals) - 1, 2)]
                    if len(vals) % 2:
                        nxt.append(vals[-1])
                    vals = nxt
                return vals[0]

            mx = _tree(s, jnp.maximum)
            e = tuple(jnp.exp(s[m] - mx) for m in range(M_SLOTS))
            den = _tree(e, lambda a, b: a + b)
            r = jnp.float32(1.0) / den
            p = tuple(e[m] * r for m in range(M_SLOTS))

            def av_d(i, _):
                for u in range(4):
                    d = i * 4 + u
                    terms = [p[m] * v_v[d, pl.ds(kbase + m, 16)]
                             for m in range(M_SLOTS)]
                    o_v[d, pl.ds(base, 16)] = _tree(terms, lambda a, b: a + b)
                return 0

            lax.fori_loop(0, D_K // 4, av_d, 0)
            return 0

        lax.fori_loop(0, CH // 16, group_body, 0)
        pltpu.sync_copy(o_v, att_hbm.at[pl.ds(row0, D_K), pl.ds(local * CH, CH)])
        return 0

    lax.fori_loop(0, NCH // 4, chunk_body, 0)


def _out_kernel(att_ref, x_ref, wf_ref, bf_ref, g_ref, b_ref, out_ref):
    att_t = att_ref[...]
    x = x_ref[...]
    tn = (((0,), (0,)), ((), ()))
    ctx = lax.dot_general(att_t, wf_ref[...], tn,
                          preferred_element_type=jnp.float32)
    ctx = ctx + bf_ref[...] + x
    mu = jnp.mean(ctx, axis=1, keepdims=True)
    d = ctx - mu
    var = jnp.mean(d * d, axis=1, keepdims=True)
    out_ref[...] = d * lax.rsqrt(var + jnp.float32(EPS)) * g_ref[...] + b_ref[...]


@jax.jit
def kernel(hidden_states, w_qs, w_ks, w_vs, w_fc, b_fc, gamma, beta,
           q_k_mask):
    del q_k_mask
    xp = jnp.pad(hidden_states, ((0, 0), (PADL, PADL), (0, 0)))
    bf = b_fc.reshape(1, D_MODEL)
    g = gamma.reshape(1, D_MODEL)
    b = beta.reshape(1, D_MODEL)
    wqt, wkt, wvt = w_qs.T, w_ks.T, w_vs.T

    R1 = D_MODEL // 4
    sc_calls = [
        pl.kernel(
            functools.partial(_sc_attention, half=half),
            out_type=jax.ShapeDtypeStruct((D_MODEL, SEQ // 2), jnp.float32),
            mesh=plsc.VectorSubcoreMesh(core_axis_name="c",
                                        subcore_axis_name="s"),
            compiler_params=pltpu.CompilerParams(use_tc_tiling_on_sc=False),
            scratch_types=[
                pltpu.VMEM((D_K, CH), jnp.float32),
                pltpu.VMEM((D_K, KW), jnp.float32),
                pltpu.VMEM((D_K, KW), jnp.float32),
                pltpu.VMEM((D_K, CH), jnp.float32),
            ],
        )
        for half in range(2)
    ]

    outs = []
    for bi in range(BATCH):
        qt, kt, vt = pl.pallas_call(
            _proj_kernel,
            grid=(D_MODEL // R1,),
            in_specs=[
                pl.BlockSpec((SP, D_MODEL), lambda t: (0, 0)),
                pl.BlockSpec((R1, D_MODEL), lambda t: (t, 0)),
                pl.BlockSpec((R1, D_MODEL), lambda t: (t, 0)),
                pl.BlockSpec((R1, D_MODEL), lambda t: (t, 0)),
            ],
            out_specs=[
                pl.BlockSpec((R1, SP), lambda t: (t, 0)),
                pl.BlockSpec((R1, SP), lambda t: (t, 0)),
                pl.BlockSpec((R1, SP), lambda t: (t, 0)),
            ],
            out_shape=[jax.ShapeDtypeStruct((D_MODEL, SP), jnp.float32)] * 3,
        )(xp[bi], wqt, wkt, wvt)

        halves = []
        for hf in range(2):
            att_t = sc_calls[hf](qt, kt, vt)
            out_h = pl.pallas_call(
                _out_kernel,
                grid=(SEQ // 2 // T2,),
                in_specs=[
                    pl.BlockSpec((D_MODEL, T2), lambda t: (0, t)),
                    pl.BlockSpec((T2, D_MODEL), lambda t: (t, 0)),
                    pl.BlockSpec((D_MODEL, D_MODEL), lambda t: (0, 0)),
                    pl.BlockSpec((1, D_MODEL), lambda t: (0, 0)),
                    pl.BlockSpec((1, D_MODEL), lambda t: (0, 0)),
                    pl.BlockSpec((1, D_MODEL), lambda t: (0, 0)),
                ],
                out_specs=pl.BlockSpec((T2, D_MODEL), lambda t: (t, 0)),
                out_shape=jax.ShapeDtypeStruct((SEQ // 2, D_MODEL),
                                               jnp.float32),
            )(att_t, hidden_states[bi, hf * (SEQ // 2):(hf + 1) * (SEQ // 2)],
              w_fc, bf, g, b)
            halves.append(out_h)
        outs.append(jnp.concatenate(halves, axis=0))
    return jnp.stack(outs, axis=0)


# speedup vs baseline: 1.0478x; 1.0478x over previous
import functools

import jax
import jax.numpy as jnp
from jax import lax
from jax.experimental import pallas as pl
from jax.experimental.pallas import tpu as pltpu
from jax.experimental.pallas import tpu_sc as plsc

BATCH = 2
SEQ = 2048
D_MODEL = 1024
N_HEAD = 16
D_K = 64
WIN = 8
M_SLOTS = 2 * WIN + 1
EPS = 1e-6

PADL = 64
SP = SEQ + 2 * PADL
T2 = SEQ // 4
CH = 256
KW = CH + 2 * WIN
NCH = SEQ // CH


def _proj_kernel(x_ref, wqt_ref, wkt_ref, wvt_ref, qt_ref, kt_ref, vt_ref):
    x = x_ref[...]
    nt = (((1,), (1,)), ((), ()))
    q = lax.dot_general(wqt_ref[...], x, nt, preferred_element_type=jnp.float32)
    qt_ref[...] = q * jnp.float32(1.0 / 8.0)
    kt_ref[...] = lax.dot_general(wkt_ref[...], x, nt,
                                  preferred_element_type=jnp.float32)
    vt_ref[...] = lax.dot_general(wvt_ref[...], x, nt,
                                  preferred_element_type=jnp.float32)


def _sc_attention(qt_hbm, kt_hbm, vt_hbm, att_hbm, q_v, k_v, v_v, o_v):
    wid = lax.axis_index("s") * 2 + lax.axis_index("c")
    half = wid % 2
    h = wid // 2
    row0 = h * D_K

    def chunk_body(c, _):
        chunk = half * (NCH // 2) + c
        c0 = PADL + chunk * CH
        pltpu.sync_copy(qt_hbm.at[pl.ds(row0, D_K), pl.ds(c0, CH)], q_v)
        pltpu.sync_copy(kt_hbm.at[pl.ds(row0, D_K), pl.ds(c0 - WIN, KW)], k_v)
        pltpu.sync_copy(vt_hbm.at[pl.ds(row0, D_K), pl.ds(c0 - WIN, KW)], v_v)

        def group_body(g, _):
            base = g * 16
            kbase = g * 16

            def score_d(i, accs):
                for u in range(4):
                    d = i * 4 + u
                    qv = q_v[d, pl.ds(base, 16)]
                    accs = tuple(
                        accs[m] + qv * k_v[d, pl.ds(kbase + m, 16)]
                        for m in range(M_SLOTS))
                return accs

            zeros = tuple(jnp.zeros((16,), jnp.float32) for _ in range(M_SLOTS))
            s = lax.fori_loop(0, D_K // 4, score_d, zeros)

            def _tree(vals, op):
                vals = list(vals)
                while len(vals) > 1:
                    nxt = [op(vals[i], vals[i + 1])
                           for i in range(0, len(vals) - 1, 2)]
                    if len(vals) % 2:
                        nxt.append(vals[-1])
                    vals = nxt
                return vals[0]

            mx = _tree(s, jnp.maximum)
            e = tuple(jnp.exp(s[m] - mx) for m in range(M_SLOTS))
            den = _tree(e, lambda a, b: a + b)
            r = jnp.float32(1.0) / den
            p = tuple(e[m] * r for m in range(M_SLOTS))

            def av_d(i, _):
                for u in range(4):
                    d = i * 4 + u
                    terms = [p[m] * v_v[d, pl.ds(kbase + m, 16)]
                             for m in range(M_SLOTS)]
                    o_v[d, pl.ds(base, 16)] = _tree(terms, lambda a, b: a + b)
                return 0

            lax.fori_loop(0, D_K // 4, av_d, 0)
            return 0

        lax.fori_loop(0, CH // 16, group_body, 0)
        pltpu.sync_copy(o_v, att_hbm.at[pl.ds(row0, D_K), pl.ds(chunk * CH, CH)])
        return 0

    lax.fori_loop(0, NCH // 2, chunk_body, 0)


def _out_kernel(att_ref, x_ref, wf_ref, bf_ref, g_ref, b_ref, out_ref):
    att_t = att_ref[...]
    x = x_ref[...]
    tn = (((0,), (0,)), ((), ()))
    ctx = lax.dot_general(att_t, wf_ref[...], tn,
                          preferred_element_type=jnp.float32)
    ctx = ctx + bf_ref[...] + x
    mu = jnp.mean(ctx, axis=1, keepdims=True)
    d = ctx - mu
    var = jnp.mean(d * d, axis=1, keepdims=True)
    out_ref[...] = d * lax.rsqrt(var + jnp.float32(EPS)) * g_ref[...] + b_ref[...]


@jax.jit
def kernel(hidden_states, w_qs, w_ks, w_vs, w_fc, b_fc, gamma, beta,
           q_k_mask):
    del q_k_mask
    xp = jnp.pad(hidden_states, ((0, 0), (PADL, PADL), (0, 0)))
    bf = b_fc.reshape(1, D_MODEL)
    g = gamma.reshape(1, D_MODEL)
    b = beta.reshape(1, D_MODEL)
    wqt, wkt, wvt = w_qs.T, w_ks.T, w_vs.T

    R1 = D_MODEL // 4
    sc_call = pl.kernel(
        _sc_attention,
        out_type=jax.ShapeDtypeStruct((D_MODEL, SEQ), jnp.float32),
        mesh=plsc.VectorSubcoreMesh(core_axis_name="c", subcore_axis_name="s"),
        compiler_params=pltpu.CompilerParams(use_tc_tiling_on_sc=False),
        scratch_types=[
            pltpu.VMEM((D_K, CH), jnp.float32),
            pltpu.VMEM((D_K, KW), jnp.float32),
            pltpu.VMEM((D_K, KW), jnp.float32),
            pltpu.VMEM((D_K, CH), jnp.float32),
        ],
    )

    outs = []
    for bi in range(BATCH):
        qt, kt, vt = pl.pallas_call(
            _proj_kernel,
            grid=(D_MODEL // R1,),
            in_specs=[
                pl.BlockSpec((SP, D_MODEL), lambda t: (0, 0)),
                pl.BlockSpec((R1, D_MODEL), lambda t: (t, 0)),
                pl.BlockSpec((R1, D_MODEL), lambda t: (t, 0)),
                pl.BlockSpec((R1, D_MODEL), lambda t: (t, 0)),
            ],
            out_specs=[
                pl.BlockSpec((R1, SP), lambda t: (t, 0)),
                pl.BlockSpec((R1, SP), lambda t: (t, 0)),
                pl.BlockSpec((R1, SP), lambda t: (t, 0)),
            ],
            out_shape=[jax.ShapeDtypeStruct((D_MODEL, SP), jnp.float32)] * 3,
        )(xp[bi], wqt, wkt, wvt)

        att_t = sc_call(qt, kt, vt)

        out_b = pl.pallas_call(
            _out_kernel,
            grid=(SEQ // T2,),
            in_specs=[
                pl.BlockSpec((D_MODEL, T2), lambda t: (0, t)),
                pl.BlockSpec((T2, D_MODEL), lambda t: (t, 0)),
                pl.BlockSpec((D_MODEL, D_MODEL), lambda t: (0, 0)),
                pl.BlockSpec((1, D_MODEL), lambda t: (0, 0)),
                pl.BlockSpec((1, D_MODEL), lambda t: (0, 0)),
                pl.BlockSpec((1, D_MODEL), lambda t: (0, 0)),
            ],
            out_specs=pl.BlockSpec((T2, D_MODEL), lambda t: (t, 0)),
            out_shape=jax.ShapeDtypeStruct((SEQ, D_MODEL), jnp.float32),
        )(att_t, hidden_states[bi], w_fc, bf, g, b)
        outs.append(out_b)
    return jnp.stack(outs, axis=0)
